# Initial kernel scaffold; baseline (speedup 1.0000x reference)
import jax
import jax.numpy as jnp
from jax.experimental import pallas as pl


def kernel(x, noise, ln1_g, ln1_b, Wq, Wk, Wv, Wp, bp, ln2_g, ln2_b, Wr, br, Wn, bn, temp, dW1, dB1, dW2, dB2, dW3, dB3, dLg, dLb, sW1, sB1, sW2, sB2, sLg, sLb):
    raise NotImplementedError("write your pallas kernel here")


# speedup vs baseline: 2.5751x; 2.5751x over previous
import functools

import jax
import jax.numpy as jnp
from jax.experimental import pallas as pl
from jax.experimental.pallas import tpu as pltpu

B, T, C, H, D, E, K, FF = 128, 128, 192, 6, 32, 6, 2, 768
N = B * T
NEG = -1e30


def _ln_f(x, g, b):
    m = jnp.mean(x, axis=-1, keepdims=True)
    v = jnp.mean((x - m) ** 2, axis=-1, keepdims=True)
    return (x - m) * jax.lax.rsqrt(v + 1e-5) * g + b


def _gelu_f(x):
    return 0.5 * x * (1.0 + jax.lax.erf(x * (2.0 ** -0.5)))


def _softplus_f(x):
    return jnp.maximum(x, 0.0) + jnp.log1p(jnp.exp(-jnp.abs(x)))


def _attn_kernel(x_ref, ln1g, ln1b, wq, wk, wv, wp, bp, ln2g, ln2b,
                 h_ref, y_ref):
    x = x_ref[0]
    xln = _ln_f(x, ln1g[...], ln1b[...])
    q = jnp.dot(xln, wq[...], preferred_element_type=jnp.float32)
    k = jnp.dot(xln, wk[...], preferred_element_type=jnp.float32)
    v = jnp.dot(xln, wv[...], preferred_element_type=jnp.float32)
    scale = C ** -0.5
    row = jax.lax.broadcasted_iota(jnp.int32, (T, T), 0)
    col = jax.lax.broadcasted_iota(jnp.int32, (T, T), 1)
    causal = row >= col
    outs = []
    for hh in range(H):
        qh = q[:, hh * D:(hh + 1) * D]
        kh = k[:, hh * D:(hh + 1) * D]
        vh = v[:, hh * D:(hh + 1) * D]
        s = jax.lax.dot_general(qh, kh, (((1,), (1,)), ((), ())),
                                preferred_element_type=jnp.float32) * scale
        s = jnp.where(causal, s, -jnp.inf)
        mx = jnp.max(s, axis=1, keepdims=True)
        p = jnp.exp(s - mx)
        p = p / jnp.sum(p, axis=1, keepdims=True)
        outs.append(jnp.dot(p, vh, preferred_element_type=jnp.float32))
    o = jnp.concatenate(outs, axis=1)
    o = jnp.dot(o, wp[...], preferred_element_type=jnp.float32) + bp[...] + x
    h_ref[0] = o
    y_ref[0] = _ln_f(o, ln2g[...], ln2b[...])


def _run_attn(x, ln1_g, ln1_b, Wq, Wk, Wv, Wp, bp, ln2_g, ln2_b):
    wq2 = Wq.transpose(1, 0, 2).reshape(C, H * D)
    wk2 = Wk.transpose(1, 0, 2).reshape(C, H * D)
    wv2 = Wv.transpose(1, 0, 2).reshape(C, H * D)
    full = lambda shp: pl.BlockSpec(shp, lambda i: (0,) * len(shp))
    h, y = pl.pallas_call(
        _attn_kernel,
        grid=(B,),
        in_specs=[
            pl.BlockSpec((1, T, C), lambda i: (i, 0, 0)),
            full((1, C)), full((1, C)),
            full((C, H * D)), full((C, H * D)), full((C, H * D)),
            full((C, C)), full((1, C)),
            full((1, C)), full((1, C)),
        ],
        out_specs=[
            pl.BlockSpec((1, T, C), lambda i: (i, 0, 0)),
            pl.BlockSpec((1, T, C), lambda i: (i, 0, 0)),
        ],
        out_shape=[
            jax.ShapeDtypeStruct((B, T, C), jnp.float32),
            jax.ShapeDtypeStruct((B, T, C), jnp.float32),
        ],
    )(x, ln1_g.reshape(1, C), ln1_b.reshape(1, C), wq2, wk2, wv2,
      Wp, bp.reshape(1, C), ln2_g.reshape(1, C), ln2_b.reshape(1, C))
    return h, y


BT_R = 512
E8 = 8


def _route_kernel(y_ref, nct_ref, wr_ref, br_ref, wn_ref, bn_ref,
                  meta_ref, gate8_ref, counts_ref, carry_ref):
    i = pl.program_id(0)

    @pl.when(i == 0)
    def _():
        carry_ref[...] = jnp.zeros((1, E8), jnp.float32)

    y = y_ref[...]
    logits = jnp.dot(y, wr_ref[...], preferred_element_type=jnp.float32) + br_ref[...]
    nl = jnp.dot(y, wn_ref[...], preferred_element_type=jnp.float32) + bn_ref[...]
    noisy = logits + nct_ref[...] * _softplus_f(nl)

    iota8 = jax.lax.broadcasted_iota(jnp.int32, (BT_R, E8), 1)
    i1 = jnp.argmax(noisy, axis=1).astype(jnp.int32)
    v1 = jnp.max(noisy, axis=1)
    m1 = iota8 == i1[:, None]
    noisy2 = jnp.where(m1, -jnp.inf, noisy)
    i2 = jnp.argmax(noisy2, axis=1).astype(jnp.int32)
    v2 = jnp.max(noisy2, axis=1)
    m2 = iota8 == i2[:, None]
    g1 = 1.0 / (1.0 + jnp.exp(v2 - v1))
    g2 = 1.0 - g1

    onehot = (m1 | m2).astype(jnp.float32)
    gate8_ref[...] = (jnp.where(m1, g1[:, None], 0.0)
                      + jnp.where(m2, g2[:, None], 0.0))

    rr = jax.lax.broadcasted_iota(jnp.int32, (BT_R, BT_R), 0)
    cc = jax.lax.broadcasted_iota(jnp.int32, (BT_R, BT_R), 1)
    tril = (rr > cc).astype(jnp.float32)
    before = jax.lax.dot_general(tril, onehot, (((1,), (0,)), ((), ())),
                                 preferred_element_type=jnp.float32)
    base = before + carry_ref[...]
    r1 = jnp.sum(jnp.where(m1, base, 0.0), axis=1)
    r2 = jnp.sum(jnp.where(m2, base, 0.0), axis=1)
    new_carry = carry_ref[...] + jnp.sum(onehot, axis=0, keepdims=True)
    carry_ref[...] = new_carry
    counts_ref[...] = new_carry

    meta_ref[...] = jnp.concatenate(
        [i1[:, None].astype(jnp.float32), i2[:, None].astype(jnp.float32),
         r1[:, None], r2[:, None], g1[:, None], g2[:, None],
         jnp.zeros((BT_R, 2), jnp.float32)], axis=1)


def _run_route(y_flat, nct8, Wr, br, Wn, bn):
    wr8 = jnp.zeros((C, E8), jnp.float32).at[:, :E].set(Wr)
    br8 = jnp.full((1, E8), NEG, jnp.float32).at[0, :E].set(br)
    wn8 = jnp.zeros((C, E8), jnp.float32).at[:, :E].set(Wn)
    bn8 = jnp.zeros((1, E8), jnp.float32).at[0, :E].set(bn)
    full = lambda shp: pl.BlockSpec(shp, lambda i: (0,) * len(shp))
    meta, gate8, counts = pl.pallas_call(
        _route_kernel,
        grid=(N // BT_R,),
        in_specs=[
            pl.BlockSpec((BT_R, C), lambda i: (i, 0)),
            pl.BlockSpec((BT_R, E8), lambda i: (i, 0)),
            full((C, E8)), full((1, E8)), full((C, E8)), full((1, E8)),
        ],
        out_specs=[
            pl.BlockSpec((BT_R, E8), lambda i: (i, 0)),
            pl.BlockSpec((BT_R, E8), lambda i: (i, 0)),
            pl.BlockSpec((1, E8), lambda i: (0, 0)),
        ],
        out_shape=[
            jax.ShapeDtypeStruct((N, E8), jnp.float32),
            jax.ShapeDtypeStruct((N, E8), jnp.float32),
            jax.ShapeDtypeStruct((1, E8), jnp.float32),
        ],
        scratch_shapes=[pltpu.VMEM((1, E8), jnp.float32)],
    )(y_flat, nct8, wr8, br8, wn8, bn8)
    return meta, gate8, counts


BT_E = 256


def _expert_kernel(y_ref, h_ref, gate_ref,
                   dW1, dB1, dW2, dB2, dW3, dB3, dLg, dLb,
                   sW1, sB1, sW2, sB2, sLg, sLb, out_ref):
    x = y_ref[...]
    gate = gate_ref[...]
    acc = h_ref[...]
    for e in range(2):
        h1 = _gelu_f(jnp.dot(x, dW1[e], preferred_element_type=jnp.float32)
                     + dB1[e][None, :])
        h2 = _gelu_f(jnp.dot(h1, dW2[e], preferred_element_type=jnp.float32)
                     + dB2[e][None, :])
        h3 = jnp.dot(h2, dW3[e], preferred_element_type=jnp.float32) + dB3[e][None, :]
        u = _ln_f(x + h3, dLg[e][None, :], dLb[e][None, :])
        acc = acc + gate[:, e:e + 1] * u
    for e in range(4):
        h1 = _gelu_f(jnp.dot(x, sW1[e], preferred_element_type=jnp.float32)
                     + sB1[e][None, :])
        h3 = jnp.dot(h1, sW2[e], preferred_element_type=jnp.float32) + sB2[e][None, :]
        u = _ln_f(x + h3, sLg[e][None, :], sLb[e][None, :])
        acc = acc + gate[:, 2 + e:3 + e] * u
    out_ref[...] = acc


def _run_experts(y_flat, h_flat, gate8,
                 dW1, dB1, dW2, dB2, dW3, dB3, dLg, dLb,
                 sW1, sB1, sW2, sB2, sLg, sLb):
    full = lambda shp: pl.BlockSpec(shp, lambda i: (0,) * len(shp))
    out = pl.pallas_call(
        _expert_kernel,
        grid=(N // BT_E,),
        in_specs=[
            pl.BlockSpec((BT_E, C), lambda i: (i, 0)),
            pl.BlockSpec((BT_E, C), lambda i: (i, 0)),
            pl.BlockSpec((BT_E, E8), lambda i: (i, 0)),
            full((2, C, FF)), full((2, FF)), full((2, FF, FF)), full((2, FF)),
            full((2, FF, C)), full((2, C)), full((2, C)), full((2, C)),
            full((4, C, FF)), full((4, FF)), full((4, FF, C)), full((4, C)),
            full((4, C)), full((4, C)),
        ],
        out_specs=pl.BlockSpec((BT_E, C), lambda i: (i, 0)),
        out_shape=jax.ShapeDtypeStruct((N, C), jnp.float32),
    )(y_flat, h_flat, gate8,
      dW1, dB1, dW2, dB2, dW3, dB3, dLg, dLb,
      sW1, sB1, sW2, sB2, sLg, sLb)
    return out


def kernel(x, noise, ln1_g, ln1_b, Wq, Wk, Wv, Wp, bp, ln2_g, ln2_b,
           Wr, br, Wn, bn, temp,
           dW1, dB1, dW2, dB2, dW3, dB3, dLg, dLb,
           sW1, sB1, sW2, sB2, sLg, sLb):
    h, y = _run_attn(x, ln1_g, ln1_b, Wq, Wk, Wv, Wp, bp, ln2_g, ln2_b)
    y_flat = y.reshape(N, C)
    h_flat = h.reshape(N, C)

    ct = jnp.clip(temp, 0.5, 2.0)
    nct8 = jnp.zeros((N, E8), jnp.float32).at[:, :E].set(
        ct * noise.reshape(N, E))

    meta, gate8, counts = _run_route(y_flat, nct8, Wr, br, Wn, bn)
    del meta, counts

    out = _run_experts(y_flat, h_flat, gate8,
                       dW1, dB1, dW2, dB2, dW3, dB3, dLg, dLb,
                       sW1, sB1, sW2, sB2, sLg, sLb)
    return out.reshape(B, T, C)
